# baseline (speedup 1.0000x reference)
import functools

import jax
import jax.numpy as jnp
from jax import lax
from jax.experimental import pallas as pl
from jax.experimental.pallas import tpu as pltpu
from jax.experimental.pallas import tpu_sc as plsc

D_MODEL = 1024
NUM_ROWS = 256
NUM_CORES = 2
NUM_SUBCORES = 16
NUM_WORKERS = NUM_CORES * NUM_SUBCORES
LANES = 16
PACKED_W = D_MODEL // 2


def _ln_table_body(table_ref, gamma_ref, beta_ref, out_ref):
    x = table_ref[...]
    mean = jnp.mean(x, axis=1, keepdims=True)
    cent = x - mean
    var = jnp.mean(cent * cent, axis=1, keepdims=True)
    inv = lax.rsqrt(var + 1e-5)
    out_ref[...] = cent * inv * gamma_ref[...] + beta_ref[...]


def _normalize_table(table, gamma, beta):
    return pl.pallas_call(
        _ln_table_body,
        out_shape=jax.ShapeDtypeStruct((NUM_ROWS, D_MODEL), jnp.float32),
    )(table, gamma.reshape(1, D_MODEL), beta.reshape(1, D_MODEL))


def _pack_table(tab_n):
    perm = tab_n.reshape(NUM_ROWS, D_MODEL // 32, 2, 16)
    perm = perm.transpose(0, 1, 3, 2).reshape(NUM_ROWS, D_MODEL)
    bf = perm.astype(jnp.bfloat16).reshape(NUM_ROWS, PACKED_W, 2)
    return jax.lax.bitcast_convert_type(bf, jnp.int32)


def _make_sc_gather(total_tokens):
    assert total_tokens % (8 * NUM_WORKERS) == 0
    tokens_per_worker = total_tokens // NUM_WORKERS
    chunk = 32
    n_chunks = tokens_per_worker // chunk
    assert n_chunks % 2 == 0 and n_chunks >= 4
    mesh = plsc.VectorSubcoreMesh(
        core_axis_name="c",
        subcore_axis_name="s",
        num_cores=NUM_CORES,
        num_subcores=NUM_SUBCORES,
    )

    @functools.partial(
        pl.kernel,
        out_type=jax.ShapeDtypeStruct((total_tokens, D_MODEL), jnp.int32),
        mesh=mesh,
        scratch_types=[
            pltpu.VMEM((tokens_per_worker,), jnp.int32),
            pltpu.VMEM((chunk, PACKED_W), jnp.int32),
            pltpu.VMEM((chunk, PACKED_W), jnp.int32),
            pltpu.VMEM((chunk, D_MODEL), jnp.int32),
            pltpu.VMEM((chunk, D_MODEL), jnp.int32),
            pltpu.SemaphoreType.DMA,
            pltpu.SemaphoreType.DMA,
            pltpu.SemaphoreType.DMA,
            pltpu.SemaphoreType.DMA,
        ],
    )
    def sc_gather(tab_hbm, idx_hbm, out_hbm, idx_v, pk0, pk1, st0, st1, g0, g1, s0, s1):
        wid = lax.axis_index("s") * NUM_CORES + lax.axis_index("c")
        base = wid * tokens_per_worker
        pltpu.sync_copy(idx_hbm.at[pl.ds(base, tokens_per_worker)], idx_v)

        def clamp_body(i, carry):
            v = idx_v[pl.ds(i * LANES, LANES)]
            idx_v[pl.ds(i * LANES, LANES)] = jnp.clip(v, 0, NUM_ROWS - 1)
            return carry

        lax.fori_loop(0, tokens_per_worker // LANES, clamp_body, 0)

        def start_gather(ci, pk, sem):
            pltpu.async_copy(tab_hbm.at[idx_v.at[pl.ds(ci * chunk, chunk)]], pk, sem)

        def start_scatter(ci, st, sem):
            pltpu.async_copy(st, out_hbm.at[pl.ds(base + ci * chunk, chunk)], sem)

        def wait_gather(pk, sem):
            pltpu.make_async_copy(tab_hbm.at[pl.ds(0, chunk)], pk, sem).wait()

        def wait_scatter(st, sem):
            pltpu.make_async_copy(st, out_hbm.at[pl.ds(base, chunk)], sem).wait()

        def convert(pk, st):
            def row_body(i, carry):
                refs = [(pk.at[2 * i], st.at[2 * i]), (pk.at[2 * i + 1], st.at[2 * i + 1])]
                for pkr, str_ in refs:
                    for b in range(2):
                        ws = [
                            pkr[pl.ds((16 * b + j) * LANES, LANES)]
                            for j in range(16)
                        ]
                        for j, w in enumerate(ws):
                            str_[pl.ds(32 * (16 * b + j), LANES)] = w << 16
                        for j, w in enumerate(ws):
                            str_[pl.ds(32 * (16 * b + j) + LANES, LANES)] = (
                                w & jnp.int32(-65536)
                            )
                return carry

            lax.fori_loop(0, chunk // 2, row_body, 0)

        start_gather(0, pk0, g0)
        start_gather(1, pk1, g1)

        wait_gather(pk0, g0)
        convert(pk0, st0)
        start_scatter(0, st0, s0)
        start_gather(2, pk0, g0)
        wait_gather(pk1, g1)
        convert(pk1, st1)
        start_scatter(1, st1, s1)
        start_gather(3, pk1, g1)

        def pair_body(p, carry):
            ci = p * 2
            wait_scatter(st0, s0)
            wait_gather(pk0, g0)
            convert(pk0, st0)
            start_scatter(ci, st0, s0)
            start_gather(ci + 2, pk0, g0)
            wait_scatter(st1, s1)
            wait_gather(pk1, g1)
            convert(pk1, st1)
            start_scatter(ci + 1, st1, s1)
            start_gather(ci + 3, pk1, g1)
            return carry

        lax.fori_loop(1, n_chunks // 2 - 1, pair_body, 0)

        last = n_chunks - 2
        wait_scatter(st0, s0)
        wait_gather(pk0, g0)
        convert(pk0, st0)
        start_scatter(last, st0, s0)
        wait_scatter(st1, s1)
        wait_gather(pk1, g1)
        convert(pk1, st1)
        start_scatter(last + 1, st1, s1)
        wait_scatter(st0, s0)
        wait_scatter(st1, s1)

    return sc_gather


def kernel(byte_ids, table, gamma, beta):
    batch, seq = byte_ids.shape
    total = batch * seq
    ids_flat = byte_ids.reshape(total).astype(jnp.int32)
    tab_packed = _pack_table(_normalize_table(table, gamma, beta))
    out = _make_sc_gather(total)(tab_packed, ids_flat)
    out = jax.lax.bitcast_convert_type(out, jnp.float32)
    return out.reshape(batch, seq, D_MODEL)


# speedup vs baseline: 1.4078x; 1.4078x over previous
import functools

import jax
import jax.numpy as jnp
from jax import lax
from jax.experimental import pallas as pl
from jax.experimental.pallas import tpu as pltpu
from jax.experimental.pallas import tpu_sc as plsc

D_MODEL = 1024
NUM_ROWS = 256
NUM_CORES = 2
NUM_SUBCORES = 16
NUM_WORKERS = NUM_CORES * NUM_SUBCORES
NBUF = 3


def _ln_table_body(table_ref, gamma_ref, beta_ref, out_ref):
    x = table_ref[...]
    mean = jnp.mean(x, axis=1, keepdims=True)
    cent = x - mean
    var = jnp.mean(cent * cent, axis=1, keepdims=True)
    inv = lax.rsqrt(var + 1e-5)
    out_ref[...] = cent * inv * gamma_ref[...] + beta_ref[...]


def _normalize_table(table, gamma, beta):
    return pl.pallas_call(
        _ln_table_body,
        out_shape=jax.ShapeDtypeStruct((NUM_ROWS, D_MODEL), jnp.float32),
    )(table, gamma.reshape(1, D_MODEL), beta.reshape(1, D_MODEL))


def _make_sc_gather(total_tokens):
    assert total_tokens % (8 * NUM_WORKERS) == 0
    tokens_per_worker = total_tokens // NUM_WORKERS
    chunk = 32
    n_chunks = tokens_per_worker // chunk
    n_rounds = n_chunks // NBUF
    n_tail = n_chunks - n_rounds * NBUF
    assert n_rounds >= 2
    mesh = plsc.VectorSubcoreMesh(
        core_axis_name="c",
        subcore_axis_name="s",
        num_cores=NUM_CORES,
        num_subcores=NUM_SUBCORES,
    )

    @functools.partial(
        pl.kernel,
        out_type=jax.ShapeDtypeStruct((total_tokens, D_MODEL), jnp.float32),
        mesh=mesh,
        scratch_types=[
            pltpu.VMEM((tokens_per_worker,), jnp.int32),
            pltpu.VMEM((NBUF, chunk, D_MODEL), jnp.float32),
            pltpu.SemaphoreType.DMA,
            pltpu.SemaphoreType.DMA,
            pltpu.SemaphoreType.DMA,
            pltpu.SemaphoreType.DMA,
            pltpu.SemaphoreType.DMA,
            pltpu.SemaphoreType.DMA,
        ],
    )
    def sc_gather(tab_hbm, idx_hbm, out_hbm, idx_v, bufs, g0, g1, g2, s0, s1, s2):
        gsem = (g0, g1, g2)
        ssem = (s0, s1, s2)
        wid = lax.axis_index("s") * NUM_CORES + lax.axis_index("c")
        base = wid * tokens_per_worker
        pltpu.sync_copy(idx_hbm.at[pl.ds(base, tokens_per_worker)], idx_v)

        def start_gather(ci, b):
            pltpu.async_copy(
                tab_hbm.at[idx_v.at[pl.ds(ci * chunk, chunk)]], bufs.at[b], gsem[b]
            )

        def start_scatter(ci, b):
            pltpu.async_copy(
                bufs.at[b], out_hbm.at[pl.ds(base + ci * chunk, chunk)], ssem[b]
            )

        def wait_gather(b):
            pltpu.make_async_copy(
                tab_hbm.at[pl.ds(0, chunk)], bufs.at[b], gsem[b]
            ).wait()

        def wait_scatter(b):
            pltpu.make_async_copy(
                bufs.at[b], out_hbm.at[pl.ds(base, chunk)], ssem[b]
            ).wait()

        for b in range(NBUF):
            start_gather(b, b)
        for b in range(NBUF):
            wait_gather(b)
            start_scatter(b, b)

        def round_body(g, carry):
            ci = g * NBUF
            for b in range(NBUF):
                wait_scatter(b)
                start_gather(ci + b, b)
            for b in range(NBUF):
                wait_gather(b)
                start_scatter(ci + b, b)
            return carry

        lax.fori_loop(1, n_rounds, round_body, 0)

        tail0 = n_rounds * NBUF
        for t in range(n_tail):
            wait_scatter(t)
            start_gather(tail0 + t, t)
        for t in range(n_tail):
            wait_gather(t)
            start_scatter(tail0 + t, t)
        for b in range(NBUF):
            wait_scatter(b)

    return sc_gather


def kernel(byte_ids, table, gamma, beta):
    batch, seq = byte_ids.shape
    total = batch * seq
    ids_flat = byte_ids.reshape(total).astype(jnp.int32)
    tab_n = _normalize_table(table, gamma, beta)
    out = _make_sc_gather(total)(tab_n, ids_flat)
    return out.reshape(batch, seq, D_MODEL)
